# baseline (device time: 20193 ns/iter reference)
import jax
import jax.numpy as jnp
from jax import lax
from jax.experimental import pallas as pl
from jax.experimental.pallas import tpu as pltpu

BLOCK_ROWS = 256
EPS = 1e-5


def kernel(x, dy, gamma):
    m, d = x.shape
    rows_per_dev = m // 2
    n_steps = rows_per_dev // BLOCK_ROWS

    def body(
        x_hbm,
        dy_hbm,
        out_ref,
        xbuf,
        dybuf,
        partial_ref,
        sum_ref,
        commx_ref,
        commy_ref,
        load_sems,
        sendx_sem,
        recvx_sem,
        sendy_sem,
        recvy_sem,
    ):
        my_x = lax.axis_index("x")
        my_y = lax.axis_index("y")
        x_nbr = (1 - my_x, my_y)
        y_nbr = (my_x, 1 - my_y)
        row0 = my_y * rows_per_dev

        def start_load(k, slot):
            cp_x = pltpu.make_async_copy(
                x_hbm.at[pl.ds(row0 + k * BLOCK_ROWS, BLOCK_ROWS), :],
                xbuf.at[slot],
                load_sems.at[slot, 0],
            )
            cp_dy = pltpu.make_async_copy(
                dy_hbm.at[pl.ds(row0 + k * BLOCK_ROWS, BLOCK_ROWS), :],
                dybuf.at[slot],
                load_sems.at[slot, 1],
            )
            cp_x.start()
            cp_dy.start()
            return cp_x, cp_dy

        start_load(0, 0)
        for k in range(n_steps):
            slot = k % 2
            if k + 1 < n_steps:
                start_load(k + 1, (k + 1) % 2)
            pltpu.make_async_copy(
                x_hbm.at[pl.ds(row0 + k * BLOCK_ROWS, BLOCK_ROWS), :],
                xbuf.at[slot],
                load_sems.at[slot, 0],
            ).wait()
            pltpu.make_async_copy(
                dy_hbm.at[pl.ds(row0 + k * BLOCK_ROWS, BLOCK_ROWS), :],
                dybuf.at[slot],
                load_sems.at[slot, 1],
            ).wait()

            xv = xbuf[slot]
            dyv = dybuf[slot]
            ones_d = jnp.ones((d, 1), jnp.float32)
            mu = jnp.dot(xv, ones_d, preferred_element_type=jnp.float32) * (1.0 / d)
            sq = xv * xv
            sumsq = jnp.dot(sq, ones_d, preferred_element_type=jnp.float32) * (
                1.0 / d
            )
            rstd = lax.rsqrt(sumsq - mu * mu + EPS)
            wdy = dyv * rstd
            m1 = wdy * xv
            ones_row = jnp.ones((1, BLOCK_ROWS), jnp.float32)
            term1 = jnp.dot(ones_row, m1, preferred_element_type=jnp.float32)
            w2 = jnp.concatenate(
                [(mu * rstd).reshape(1, BLOCK_ROWS), ones_row], axis=0
            )
            b2 = jnp.dot(w2, dyv, preferred_element_type=jnp.float32)
            dgdb = jnp.concatenate([term1 - b2[0:1], b2[1:2]], axis=0)
            if k == 0:
                partial_ref[:, :] = dgdb
            else:
                partial_ref[:, :] += dgdb

        barrier = pltpu.get_barrier_semaphore()
        for nbr in (x_nbr, y_nbr):
            pl.semaphore_signal(
                barrier, inc=1, device_id=nbr, device_id_type=pl.DeviceIdType.MESH
            )
        pl.semaphore_wait(barrier, 2)

        rdma_x = pltpu.make_async_remote_copy(
            src_ref=partial_ref,
            dst_ref=commx_ref,
            send_sem=sendx_sem,
            recv_sem=recvx_sem,
            device_id=x_nbr,
            device_id_type=pl.DeviceIdType.MESH,
        )
        rdma_x.start()
        rdma_x.wait()
        sum_ref[:, :] = partial_ref[:, :] + commx_ref[:, :]

        rdma_y = pltpu.make_async_remote_copy(
            src_ref=sum_ref,
            dst_ref=commy_ref,
            send_sem=sendy_sem,
            recv_sem=recvy_sem,
            device_id=y_nbr,
            device_id_type=pl.DeviceIdType.MESH,
        )
        rdma_y.start()
        rdma_y.wait()
        out_ref[:, :] = sum_ref[:, :] + commy_ref[:, :]

    return pl.pallas_call(
        body,
        out_shape=jax.ShapeDtypeStruct((2, d), jnp.float32),
        in_specs=[
            pl.BlockSpec(memory_space=pl.ANY),
            pl.BlockSpec(memory_space=pl.ANY),
        ],
        out_specs=pl.BlockSpec(memory_space=pltpu.VMEM),
        scratch_shapes=[
            pltpu.VMEM((2, BLOCK_ROWS, d), jnp.float32),
            pltpu.VMEM((2, BLOCK_ROWS, d), jnp.float32),
            pltpu.VMEM((2, d), jnp.float32),
            pltpu.VMEM((2, d), jnp.float32),
            pltpu.VMEM((2, d), jnp.float32),
            pltpu.VMEM((2, d), jnp.float32),
            pltpu.SemaphoreType.DMA((2, 2)),
            pltpu.SemaphoreType.DMA,
            pltpu.SemaphoreType.DMA,
            pltpu.SemaphoreType.DMA,
            pltpu.SemaphoreType.DMA,
        ],
        compiler_params=pltpu.CompilerParams(collective_id=0),
    )(x, dy)


# device time: 17472 ns/iter; 1.1557x vs baseline; 1.1557x over previous
import jax
import jax.numpy as jnp
from jax import lax
from jax.experimental import pallas as pl
from jax.experimental.pallas import tpu as pltpu

BLOCK_ROWS = 256
EPS = 1e-5
N_PEERS = 3
N_PHASES = 2


def kernel(x, dy, gamma):
    m, d = x.shape
    rows_per_dev = m // 2
    n_steps = rows_per_dev // BLOCK_ROWS
    ph1_steps = n_steps - 1

    def body(
        x_hbm,
        dy_hbm,
        out_ref,
        xbuf,
        dybuf,
        p1_ref,
        p2_ref,
        comm_ref,
        load_sems,
        send_sems,
        recv_sems,
    ):
        my_x = lax.axis_index("x")
        my_y = lax.axis_index("y")
        peers = [(1 - my_x, my_y), (my_x, 1 - my_y), (1 - my_x, 1 - my_y)]
        row0 = my_y * rows_per_dev

        def load_descs(k, slot):
            return (
                pltpu.make_async_copy(
                    x_hbm.at[pl.ds(row0 + k * BLOCK_ROWS, BLOCK_ROWS), :],
                    xbuf.at[slot],
                    load_sems.at[slot, 0],
                ),
                pltpu.make_async_copy(
                    dy_hbm.at[pl.ds(row0 + k * BLOCK_ROWS, BLOCK_ROWS), :],
                    dybuf.at[slot],
                    load_sems.at[slot, 1],
                ),
            )

        def xchg_descs(ph, src_ref):
            return [
                pltpu.make_async_remote_copy(
                    src_ref=src_ref,
                    dst_ref=comm_ref.at[ph, j],
                    send_sem=send_sems.at[ph, j],
                    recv_sem=recv_sems.at[ph, j],
                    device_id=peers[j],
                    device_id_type=pl.DeviceIdType.MESH,
                )
                for j in range(N_PEERS)
            ]

        barrier = pltpu.get_barrier_semaphore()
        for p in peers:
            pl.semaphore_signal(
                barrier, inc=1, device_id=p, device_id_type=pl.DeviceIdType.MESH
            )
        pl.semaphore_wait(barrier, N_PEERS)

        for cp in load_descs(0, 0):
            cp.start()
        for k in range(n_steps):
            slot = k % 2
            if k + 1 < n_steps:
                for cp in load_descs(k + 1, (k + 1) % 2):
                    cp.start()
            for cp in load_descs(k, slot):
                cp.wait()

            xv = xbuf[slot]
            dyv = dybuf[slot]
            mu = jnp.mean(xv, axis=1, keepdims=True)
            xc = xv - mu
            var = jnp.mean(xc * xc, axis=1, keepdims=True)
            xhat = xc * lax.rsqrt(var + EPS)
            dgamma = jnp.sum(dyv * xhat, axis=0)
            dbeta = jnp.sum(dyv, axis=0)
            acc = p2_ref if k >= ph1_steps else p1_ref
            if k == 0 or k == ph1_steps:
                acc[0, :] = dgamma
                acc[1, :] = dbeta
            else:
                acc[0, :] += dgamma
                acc[1, :] += dbeta

            if k == ph1_steps - 1:
                for r in xchg_descs(0, p1_ref):
                    r.start()

        xchg2 = xchg_descs(1, p2_ref)
        for r in xchg2:
            r.start()
        xchg1 = xchg_descs(0, p1_ref)
        for r in xchg1:
            r.wait_recv()
        acc1 = p1_ref[:, :] + comm_ref[0, 0] + comm_ref[0, 1] + comm_ref[0, 2]
        for r in xchg2:
            r.wait_recv()
        out_ref[:, :] = (
            acc1 + p2_ref[:, :] + comm_ref[1, 0] + comm_ref[1, 1] + comm_ref[1, 2]
        )
        for r in xchg1:
            r.wait_send()
        for r in xchg2:
            r.wait_send()

    return pl.pallas_call(
        body,
        out_shape=jax.ShapeDtypeStruct((2, d), jnp.float32),
        in_specs=[
            pl.BlockSpec(memory_space=pl.ANY),
            pl.BlockSpec(memory_space=pl.ANY),
        ],
        out_specs=pl.BlockSpec(memory_space=pltpu.VMEM),
        scratch_shapes=[
            pltpu.VMEM((2, BLOCK_ROWS, d), jnp.float32),
            pltpu.VMEM((2, BLOCK_ROWS, d), jnp.float32),
            pltpu.VMEM((2, d), jnp.float32),
            pltpu.VMEM((2, d), jnp.float32),
            pltpu.VMEM((N_PHASES, N_PEERS, 2, d), jnp.float32),
            pltpu.SemaphoreType.DMA((2, 2)),
            pltpu.SemaphoreType.DMA((N_PHASES, N_PEERS)),
            pltpu.SemaphoreType.DMA((N_PHASES, N_PEERS)),
        ],
        compiler_params=pltpu.CompilerParams(collective_id=0),
    )(x, dy)


# device time: 16785 ns/iter; 1.2030x vs baseline; 1.0409x over previous
import jax
import jax.numpy as jnp
from jax import lax
from jax.experimental import pallas as pl
from jax.experimental.pallas import tpu as pltpu

BLOCK_ROWS = 512
EPS = 1e-5
N_PEERS = 3
N_PHASES = 2


def kernel(x, dy, gamma):
    m, d = x.shape
    rows_per_dev = m // 2
    n_steps = rows_per_dev // BLOCK_ROWS
    ph1_steps = n_steps - 1

    def body(
        x_hbm,
        dy_hbm,
        out_ref,
        xbuf,
        dybuf,
        p1_ref,
        p2_ref,
        comm_ref,
        load_sems,
        send_sems,
        recv_sems,
    ):
        my_x = lax.axis_index("x")
        my_y = lax.axis_index("y")
        peers = [(1 - my_x, my_y), (my_x, 1 - my_y), (1 - my_x, 1 - my_y)]
        row0 = my_y * rows_per_dev

        def load_descs(k, slot):
            return (
                pltpu.make_async_copy(
                    x_hbm.at[pl.ds(row0 + k * BLOCK_ROWS, BLOCK_ROWS), :],
                    xbuf.at[slot],
                    load_sems.at[slot, 0],
                ),
                pltpu.make_async_copy(
                    dy_hbm.at[pl.ds(row0 + k * BLOCK_ROWS, BLOCK_ROWS), :],
                    dybuf.at[slot],
                    load_sems.at[slot, 1],
                ),
            )

        def xchg_descs(ph, src_ref):
            return [
                pltpu.make_async_remote_copy(
                    src_ref=src_ref,
                    dst_ref=comm_ref.at[ph, j],
                    send_sem=send_sems.at[ph, j],
                    recv_sem=recv_sems.at[ph, j],
                    device_id=peers[j],
                    device_id_type=pl.DeviceIdType.MESH,
                )
                for j in range(N_PEERS)
            ]

        for cp in load_descs(0, 0):
            cp.start()
        barrier = pltpu.get_barrier_semaphore()
        for p in peers:
            pl.semaphore_signal(
                barrier, inc=1, device_id=p, device_id_type=pl.DeviceIdType.MESH
            )
        pl.semaphore_wait(barrier, N_PEERS)
        for k in range(n_steps):
            slot = k % 2
            if k + 1 < n_steps:
                for cp in load_descs(k + 1, (k + 1) % 2):
                    cp.start()
            for cp in load_descs(k, slot):
                cp.wait()

            xv = xbuf[slot]
            dyv = dybuf[slot]
            mu = jnp.mean(xv, axis=1, keepdims=True)
            xc = xv - mu
            var = jnp.mean(xc * xc, axis=1, keepdims=True)
            xhat = xc * lax.rsqrt(var + EPS)
            dgamma = jnp.sum(dyv * xhat, axis=0)
            dbeta = jnp.sum(dyv, axis=0)
            acc = p2_ref if k >= ph1_steps else p1_ref
            if k == 0 or k == ph1_steps:
                acc[0, :] = dgamma
                acc[1, :] = dbeta
            else:
                acc[0, :] += dgamma
                acc[1, :] += dbeta

            if k == ph1_steps - 1:
                for r in xchg_descs(0, p1_ref):
                    r.start()

        xchg2 = xchg_descs(1, p2_ref)
        for r in xchg2:
            r.start()
        xchg1 = xchg_descs(0, p1_ref)
        for r in xchg1:
            r.wait_recv()
        acc1 = p1_ref[:, :] + comm_ref[0, 0] + comm_ref[0, 1] + comm_ref[0, 2]
        for r in xchg2:
            r.wait_recv()
        out_ref[:, :] = (
            acc1 + p2_ref[:, :] + comm_ref[1, 0] + comm_ref[1, 1] + comm_ref[1, 2]
        )
        for r in xchg1:
            r.wait_send()
        for r in xchg2:
            r.wait_send()

    return pl.pallas_call(
        body,
        out_shape=jax.ShapeDtypeStruct((2, d), jnp.float32),
        in_specs=[
            pl.BlockSpec(memory_space=pl.ANY),
            pl.BlockSpec(memory_space=pl.ANY),
        ],
        out_specs=pl.BlockSpec(memory_space=pltpu.VMEM),
        scratch_shapes=[
            pltpu.VMEM((2, BLOCK_ROWS, d), jnp.float32),
            pltpu.VMEM((2, BLOCK_ROWS, d), jnp.float32),
            pltpu.VMEM((2, d), jnp.float32),
            pltpu.VMEM((2, d), jnp.float32),
            pltpu.VMEM((N_PHASES, N_PEERS, 2, d), jnp.float32),
            pltpu.SemaphoreType.DMA((2, 2)),
            pltpu.SemaphoreType.DMA((N_PHASES, N_PEERS)),
            pltpu.SemaphoreType.DMA((N_PHASES, N_PEERS)),
        ],
        compiler_params=pltpu.CompilerParams(collective_id=0),
    )(x, dy)
